# baseline (speedup 1.0000x reference)
import functools

import jax
import jax.numpy as jnp
from jax import lax
from jax.experimental import pallas as pl
from jax.experimental.pallas import tpu as pltpu
from jax.experimental.pallas import tpu_sc as plsc

_B = 16384
_D = 128
_BT = 12288
_BS = _B - _BT
_NC, _NS = 2, 16
_NW = _NC * _NS
_CHUNKS = _BS // (128 * _NW)

_TROWS = 4096
_TC = _TROWS // 128

_AROWS = 2048


def _prep_body(uf_ref, d1f_ref, d2f_ref, s1_ref, s2_ref):
    uf = uf_ref[...]
    m1 = jnp.max(uf)
    m2 = jnp.max(uf - d2f_ref[...])
    s1_ref[...] = jnp.minimum(d1f_ref[...], m2)
    s2_ref[...] = jnp.minimum(d2f_ref[...], m1)


def _tc_body(uf_ref, d1f_ref, d2f_ref, v1_ref, v2_ref, o_ref):
    uf = uf_ref[...]
    m1 = jnp.max(uf)
    m2 = jnp.max(uf - d2f_ref[...])
    i = pl.program_id(0)
    d1b = d1f_ref[pl.ds(i * _TC, _TC), :]
    d2b = d2f_ref[pl.ds(i * _TC, _TC), :]
    s1t = jnp.transpose(jnp.minimum(d1b, m2))
    s2t = jnp.transpose(jnp.minimum(d2b, m1))
    for k in range(_TC):
        sl = slice(128 * k, 128 * (k + 1))
        o_ref[sl, :] = (
            v1_ref[sl, :] * s1t[:, k : k + 1] + v2_ref[sl, :] * s2t[:, k : k + 1]
        )


def _sc_body(v1_hbm, v2_hbm, s1_hbm, s2_hbm, o_hbm, v1s, v2s, os_, s1s, s2s):
    w = lax.axis_index("s") * _NC + lax.axis_index("c")
    for cc in range(_CHUNKS):
        crow = _BT // 128 + w * _CHUNKS + cc
        rbase = crow * 128
        pltpu.sync_copy(s1_hbm.at[crow], s1s)
        pltpu.sync_copy(s2_hbm.at[crow], s2s)
        pltpu.sync_copy(v1_hbm.at[pl.ds(rbase, 128)], v1s)
        pltpu.sync_copy(v2_hbm.at[pl.ds(rbase, 128)], v2s)

        for g in range(8):
            sv1 = s1s[g, :]
            sv2 = s2s[g, :]
            for t16 in range(16):
                t = 16 * g + t16
                a = jnp.full((16,), sv1[t16], jnp.float32)
                b = jnp.full((16,), sv2[t16], jnp.float32)
                for j in range(8):
                    cs = pl.ds(16 * j, 16)
                    os_[t, cs] = v1s[t, cs] * a + v2s[t, cs] * b
        pltpu.sync_copy(os_, o_hbm.at[pl.ds(rbase - _BT, 128)])


def _asm_body(sc_ref, tc_ref, o_ref):
    del tc_ref
    o_ref[...] = sc_ref[...]


def kernel(u, d1, d2, v1, v2):
    B, D = v1.shape
    uf = u.reshape(B // 128, 128)
    d1f = d1.reshape(B // 128, 128)
    d2f = d2.reshape(B // 128, 128)

    s1c, s2c = pl.pallas_call(
        _prep_body,
        out_shape=[
            jax.ShapeDtypeStruct((B // 128, 128), jnp.float32),
            jax.ShapeDtypeStruct((B // 128, 128), jnp.float32),
        ],
    )(uf, d1f, d2f)

    tc_full = pl.pallas_call(
        _tc_body,
        grid=(_BT // _TROWS,),
        in_specs=[
            pl.BlockSpec((B // 128, 128), lambda i: (0, 0)),
            pl.BlockSpec((B // 128, 128), lambda i: (0, 0)),
            pl.BlockSpec((B // 128, 128), lambda i: (0, 0)),
            pl.BlockSpec((_TROWS, D), lambda i: (i, 0)),
            pl.BlockSpec((_TROWS, D), lambda i: (i, 0)),
        ],
        out_specs=pl.BlockSpec((_TROWS, D), lambda i: (i, 0)),
        out_shape=jax.ShapeDtypeStruct((B, D), jnp.float32),
    )(uf, d1f, d2f, v1, v2)

    sc_kernel = functools.partial(
        pl.kernel,
        mesh=plsc.VectorSubcoreMesh(core_axis_name="c", subcore_axis_name="s"),
        out_type=jax.ShapeDtypeStruct((_BS, D), jnp.float32),
        scratch_types=[
            pltpu.VMEM((128, 128), jnp.float32),
            pltpu.VMEM((128, 128), jnp.float32),
            pltpu.VMEM((128, 128), jnp.float32),
            pltpu.VMEM((8, 16), jnp.float32),
            pltpu.VMEM((8, 16), jnp.float32),
        ],
    )
    sc_out = sc_kernel(_sc_body)(
        v1, v2, s1c.reshape(B // 128, 8, 16), s2c.reshape(B // 128, 8, 16)
    )

    out = pl.pallas_call(
        _asm_body,
        grid=(_BS // _AROWS,),
        in_specs=[
            pl.BlockSpec((_AROWS, D), lambda i: (i, 0)),
            pl.BlockSpec(memory_space=pl.ANY),
        ],
        out_specs=pl.BlockSpec((_AROWS, D), lambda i: (_BT // _AROWS + i, 0)),
        out_shape=jax.ShapeDtypeStruct((B, D), jnp.float32),
        input_output_aliases={1: 0},
    )(sc_out, tc_full)
    return out
